# baseline (speedup 1.0000x reference)
import functools

import jax
import jax.numpy as jnp
from jax import lax
from jax.experimental import pallas as pl
from jax.experimental.pallas import tpu as pltpu
from jax.experimental.pallas import tpu_sc as plsc

_B = 1024
_SRC = 200
_DEC = 512
_ENC = 512
_V = 1000

_L = 16
_NC = 2
_NS = 16
_NW = _NC * _NS
_GPW = _B // _L // _NW
_VCH = _V // _L
_TAIL = _V - _L


def _gate_body(dh_ref, cv_ref, w1_ref, b1_ref, w2_ref, b2_ref, p_ref):
    w1 = w1_ref[...]
    h = lax.dot_general(dh_ref[...], w1[:, :_DEC], (((1,), (1,)), ((), ())),
                        preferred_element_type=jnp.float32)
    h += lax.dot_general(cv_ref[...], w1[:, _DEC:], (((1,), (1,)), ((), ())),
                         preferred_element_type=jnp.float32)
    h = jnp.tanh(h + b1_ref[...])
    z = lax.dot_general(h, w2_ref[...], (((1,), (1,)), ((), ())),
                        preferred_element_type=jnp.float32)
    p_ref[...] = jax.nn.sigmoid(z[:, :1] + b2_ref[0, 0])


def _sc_body(attn_hbm, vocab_hbm, chars_hbm, p_hbm, out_hbm,
             vocab_0, vocab_1, out_0, out_1, attn_0, attn_1,
             chars_0, chars_1, p_0, p_1,
             sem_in0, sem_in1, sem_out0, sem_out1):
    wid = lax.axis_index("s") * _NC + lax.axis_index("c")
    iota = lax.iota(jnp.int32, _L)
    zeros = jnp.zeros((_L,), jnp.float32)
    tail_mask = iota >= (_L - (_V - _VCH * _L))

    sets = ((vocab_0, out_0, attn_0, chars_0, p_0, sem_in0, sem_out0),
            (vocab_1, out_1, attn_1, chars_1, p_1, sem_in1, sem_out1))

    in_handles = []
    for k in range(_GPW):
        base = (wid * _GPW + k) * _L
        vocab_v, _, attn_v, chars_v, p_v, sem_in, _ = sets[k]
        in_handles.append([
            pltpu.async_copy(vocab_hbm.at[pl.ds(base, _L), :], vocab_v,
                             sem_in),
            pltpu.async_copy(attn_hbm.at[pl.ds(base, _L), :], attn_v, sem_in),
            pltpu.async_copy(chars_hbm.at[pl.ds(base, _L), :], chars_v,
                             sem_in),
            pltpu.async_copy(p_hbm.at[pl.ds(base, _L)], p_v, sem_in),
        ])

    out_handles = []
    for k in range(_GPW):
        base = (wid * _GPW + k) * _L
        vocab_v, out_v, attn_v, chars_v, p_v, sem_in, sem_out = sets[k]
        for h in in_handles[k]:
            h.wait()

        pv = p_v[...]
        ratio = pv / (1.0 - pv)

        def _scat(s, c, chars_v=chars_v, attn_v=attn_v, vocab_v=vocab_v,
                  ratio=ratio):
            col = jnp.full((_L,), s, jnp.int32)
            ch = plsc.load_gather(chars_v, [iota, col])
            aw = plsc.load_gather(attn_v, [iota, col])
            plsc.addupdate_scatter(vocab_v, [iota, ch], aw * ratio)
            return c
        lax.fori_loop(0, _SRC, _scat, 0, unroll=4)

        def _row(r, c, vocab_v=vocab_v, out_v=out_v, p_v=p_v):
            r_idx = jnp.full((_L,), r, jnp.int32)
            pr = plsc.load_gather(p_v, [r_idx])
            one_m_p = 1.0 - pr

            def _ms(i, acc):
                return acc + vocab_v[r, pl.ds(i * _L, _L)]
            msum_vec = lax.fori_loop(0, _VCH, _ms, zeros, unroll=8)
            tail = vocab_v[r, pl.ds(_TAIL, _L)]
            msum_vec = msum_vec + jnp.where(tail_mask, tail, 0.0)
            msum = jnp.broadcast_to(jnp.sum(msum_vec), (_L,))
            gs = one_m_p / (one_m_p * msum + 1e-10)

            def _fin(i, c2):
                sl = pl.ds(i * _L, _L)
                out_v[r, sl] = vocab_v[r, sl] * gs
                return c2
            lax.fori_loop(0, _VCH, _fin, 0, unroll=8)
            out_v[r, pl.ds(_TAIL, _L)] = vocab_v[r, pl.ds(_TAIL, _L)] * gs
            return c
        lax.fori_loop(0, _L, _row, 0)

        out_handles.append(
            pltpu.async_copy(out_v, out_hbm.at[pl.ds(base, _L), :], sem_out))

    for h in out_handles:
        h.wait()


def kernel(decoder_hidden, context_vector, encoder_outputs, attention_weights,
           vocab_distribution, source_chars, W1, b1, W2, b2):
    del encoder_outputs

    copy_prob = pl.pallas_call(
        _gate_body,
        out_shape=jax.ShapeDtypeStruct((_B, 1), jnp.float32),
    )(decoder_hidden, context_vector, W1,
      b1.reshape(1, _DEC), jnp.pad(W2, ((0, 127), (0, 0))), b2.reshape(1, 1))

    p_flat = copy_prob.reshape(_B)
    chars = source_chars.astype(jnp.int32)

    mesh = plsc.VectorSubcoreMesh(core_axis_name="c", subcore_axis_name="s")
    sc_call = functools.partial(
        pl.kernel, mesh=mesh,
        compiler_params=pltpu.CompilerParams(use_tc_tiling_on_sc=False,
                                             needs_layout_passes=False),
        out_type=jax.ShapeDtypeStruct((_B, _V), jnp.float32),
        scratch_types=[
            pltpu.VMEM((_L, _V), jnp.float32),
            pltpu.VMEM((_L, _V), jnp.float32),
            pltpu.VMEM((_L, _V), jnp.float32),
            pltpu.VMEM((_L, _V), jnp.float32),
            pltpu.VMEM((_L, _SRC), jnp.float32),
            pltpu.VMEM((_L, _SRC), jnp.float32),
            pltpu.VMEM((_L, _SRC), jnp.int32),
            pltpu.VMEM((_L, _SRC), jnp.int32),
            pltpu.VMEM((_L,), jnp.float32),
            pltpu.VMEM((_L,), jnp.float32),
            pltpu.SemaphoreType.DMA,
            pltpu.SemaphoreType.DMA,
            pltpu.SemaphoreType.DMA,
            pltpu.SemaphoreType.DMA,
        ],
    )(_sc_body)
    final = sc_call(attention_weights, vocab_distribution, chars, p_flat)
    return final, copy_prob


# speedup vs baseline: 1.1147x; 1.1147x over previous
import functools

import jax
import jax.numpy as jnp
from jax import lax
from jax.experimental import pallas as pl
from jax.experimental.pallas import tpu as pltpu
from jax.experimental.pallas import tpu_sc as plsc

_B = 1024
_SRC = 200
_DEC = 512
_ENC = 512
_V = 1000

_L = 16
_NC = 2
_NS = 16
_NW = _NC * _NS
_GPW = _B // _L // _NW
_VP = 1024
_SP = 256


def _gate_body(dh_ref, cv_ref, w1_ref, b1_ref, w2_ref, b2_ref, p_ref):
    w1 = w1_ref[...]
    h = lax.dot_general(dh_ref[...], w1[:, :_DEC], (((1,), (1,)), ((), ())),
                        preferred_element_type=jnp.float32)
    h += lax.dot_general(cv_ref[...], w1[:, _DEC:], (((1,), (1,)), ((), ())),
                         preferred_element_type=jnp.float32)
    h = jnp.tanh(h + b1_ref[...])
    z = lax.dot_general(h, w2_ref[...], (((1,), (1,)), ((), ())),
                        preferred_element_type=jnp.float32)
    p_ref[...] = jax.nn.sigmoid(z[:, :1] + b2_ref[0, 0])


def _combine_body(vocab_ref, acc_ref, p_ref, out_ref):
    vocab = vocab_ref[...]
    acc = acc_ref[:, :_V]
    p = p_ref[...]
    vsum = jnp.sum(vocab, axis=1, keepdims=True)
    accsum = jnp.sum(acc, axis=1, keepdims=True)
    total = (1.0 - p) * vsum + p * (accsum - vsum)
    inv = 1.0 / (total + 1e-10)
    out_ref[...] = (vocab * (1.0 - 2.0 * p) + p * acc) * inv


def _sc_body(attn_hbm, vocab_hbm, chars_hbm, acc_hbm,
             vocab_0, vocab_1, attn_0, attn_1, chars_0, chars_1,
             sem_in0, sem_in1, sem_out0, sem_out1):
    wid = lax.axis_index("s") * _NC + lax.axis_index("c")
    iota = lax.iota(jnp.int32, _L)

    sets = ((vocab_0, attn_0, chars_0, sem_in0, sem_out0),
            (vocab_1, attn_1, chars_1, sem_in1, sem_out1))

    in_handles = []
    for k in range(_GPW):
        base = (wid * _GPW + k) * _L
        vocab_v, attn_v, chars_v, sem_in, _ = sets[k]
        in_handles.append([
            pltpu.async_copy(vocab_hbm.at[pl.ds(base, _L), :], vocab_v,
                             sem_in),
            pltpu.async_copy(attn_hbm.at[pl.ds(base, _L), :], attn_v, sem_in),
            pltpu.async_copy(chars_hbm.at[pl.ds(base, _L), :], chars_v,
                             sem_in),
        ])

    out_handles = []
    for k in range(_GPW):
        base = (wid * _GPW + k) * _L
        vocab_v, attn_v, chars_v, sem_in, sem_out = sets[k]
        for h in in_handles[k]:
            h.wait()

        def _scat(s, c, chars_v=chars_v, attn_v=attn_v, vocab_v=vocab_v):
            col = jnp.full((_L,), s, jnp.int32)
            ch = plsc.load_gather(chars_v, [iota, col])
            aw = plsc.load_gather(attn_v, [iota, col])
            plsc.addupdate_scatter(vocab_v, [iota, ch], aw)
            return c
        lax.fori_loop(0, _SRC, _scat, 0, unroll=8)

        out_handles.append(
            pltpu.async_copy(vocab_v, acc_hbm.at[pl.ds(base, _L), :],
                             sem_out))

    for h in out_handles:
        h.wait()


def kernel(decoder_hidden, context_vector, encoder_outputs, attention_weights,
           vocab_distribution, source_chars, W1, b1, W2, b2):
    del encoder_outputs

    attn_p = jnp.pad(attention_weights, ((0, 0), (0, _SP - _SRC)))
    vocab_p = jnp.pad(vocab_distribution, ((0, 0), (0, _VP - _V)))
    chars_p = jnp.pad(source_chars.astype(jnp.int32),
                      ((0, 0), (0, _SP - _SRC)))

    mesh = plsc.VectorSubcoreMesh(core_axis_name="c", subcore_axis_name="s")
    sc_call = functools.partial(
        pl.kernel, mesh=mesh,
        compiler_params=pltpu.CompilerParams(use_tc_tiling_on_sc=False,
                                             needs_layout_passes=False),
        out_type=jax.ShapeDtypeStruct((_B, _VP), jnp.float32),
        scratch_types=[
            pltpu.VMEM((_L, _VP), jnp.float32),
            pltpu.VMEM((_L, _VP), jnp.float32),
            pltpu.VMEM((_L, _SP), jnp.float32),
            pltpu.VMEM((_L, _SP), jnp.float32),
            pltpu.VMEM((_L, _SP), jnp.int32),
            pltpu.VMEM((_L, _SP), jnp.int32),
            pltpu.SemaphoreType.DMA,
            pltpu.SemaphoreType.DMA,
            pltpu.SemaphoreType.DMA,
            pltpu.SemaphoreType.DMA,
        ],
    )(_sc_body)
    acc = sc_call(attn_p, vocab_p, chars_p)

    copy_prob = pl.pallas_call(
        _gate_body,
        out_shape=jax.ShapeDtypeStruct((_B, 1), jnp.float32),
    )(decoder_hidden, context_vector, W1,
      b1.reshape(1, _DEC), jnp.pad(W2, ((0, 127), (0, 0))), b2.reshape(1, 1))

    final = pl.pallas_call(
        _combine_body,
        out_shape=jax.ShapeDtypeStruct((_B, _V), jnp.float32),
    )(vocab_distribution, acc, copy_prob)
    return final, copy_prob


# speedup vs baseline: 1.2102x; 1.0857x over previous
import functools

import jax
import jax.numpy as jnp
from jax import lax
from jax.experimental import pallas as pl
from jax.experimental.pallas import tpu as pltpu
from jax.experimental.pallas import tpu_sc as plsc

_B = 1024
_SRC = 200
_DEC = 512
_ENC = 512
_V = 1000

_L = 16
_NC = 2
_NS = 16
_NW = _NC * _NS
_GPW = _B // _L // _NW
_VP = 1024
_SP = 256


def _gate_body(dh_ref, cv_ref, w1_ref, b1_ref, w2_ref, b2_ref, p_ref):
    w1 = w1_ref[...]
    h = lax.dot_general(dh_ref[...], w1[:, :_DEC], (((1,), (1,)), ((), ())),
                        preferred_element_type=jnp.float32)
    h += lax.dot_general(cv_ref[...], w1[:, _DEC:], (((1,), (1,)), ((), ())),
                         preferred_element_type=jnp.float32)
    h = jnp.tanh(h + b1_ref[...])
    z = lax.dot_general(h, w2_ref[...], (((1,), (1,)), ((), ())),
                        preferred_element_type=jnp.float32)
    p_ref[...] = jax.nn.sigmoid(z[:, :1] + b2_ref[0, 0])


def _combine_body(vocab_ref, acc_ref, p_ref, out_ref):
    vocab = vocab_ref[...]
    acc = acc_ref[:, :_V]
    p = p_ref[...]
    vsum = jnp.sum(vocab, axis=1, keepdims=True)
    accsum = jnp.sum(acc, axis=1, keepdims=True)
    total = (1.0 - p) * vsum + p * (accsum - vsum)
    inv = 1.0 / (total + 1e-10)
    out_ref[...] = (vocab * (1.0 - 2.0 * p) + p * acc) * inv


def _sc_body(attn_hbm, vocab_hbm, chars_hbm, acc_hbm,
             vocab_0, vocab_1, attn_0, attn_1, chars_0, chars_1,
             sem_in0, sem_in1, sem_out0, sem_out1):
    wid = lax.axis_index("s") * _NC + lax.axis_index("c")
    iota = lax.iota(jnp.int32, _L)

    sets = ((vocab_0, attn_0, chars_0, sem_in0, sem_out0),
            (vocab_1, attn_1, chars_1, sem_in1, sem_out1))

    in_handles = []
    for k in range(_GPW):
        base = (wid * _GPW + k) * _L
        vocab_v, attn_v, chars_v, sem_in, _ = sets[k]
        in_handles.append([
            pltpu.async_copy(vocab_hbm.at[pl.ds(base, _L), :], vocab_v,
                             sem_in),
            pltpu.async_copy(attn_hbm.at[pl.ds(base, _L), :], attn_v, sem_in),
            pltpu.async_copy(chars_hbm.at[pl.ds(base, _L), :], chars_v,
                             sem_in),
        ])

    out_handles = []
    for k in range(_GPW):
        base = (wid * _GPW + k) * _L
        vocab_v, attn_v, chars_v, sem_in, sem_out = sets[k]
        for h in in_handles[k]:
            h.wait()

        def _scat(s, c, chars_v=chars_v, attn_v=attn_v, vocab_v=vocab_v):
            col = jnp.full((_L,), s, jnp.int32)
            ch = plsc.load_gather(chars_v, [iota, col])
            aw = plsc.load_gather(attn_v, [iota, col])
            plsc.addupdate_scatter(vocab_v, [iota, ch], aw)
            return c
        lax.fori_loop(0, _SRC, _scat, 0, unroll=8)

        out_handles.append(
            pltpu.async_copy(vocab_v, acc_hbm.at[pl.ds(base, _L), :],
                             sem_out))

    for h in out_handles:
        h.wait()


def kernel(decoder_hidden, context_vector, encoder_outputs, attention_weights,
           vocab_distribution, source_chars, W1, b1, W2, b2):
    del encoder_outputs

    attn_p = jnp.pad(attention_weights, ((0, 0), (0, _SP - _SRC)))
    vocab_p = jnp.pad(vocab_distribution, ((0, 0), (0, _VP - _V)))
    chars_p = jnp.pad(source_chars.astype(jnp.int32),
                      ((0, 0), (0, _SP - _SRC)))

    mesh = plsc.VectorSubcoreMesh(core_axis_name="c", subcore_axis_name="s")
    sc_call = functools.partial(
        pl.kernel, mesh=mesh,
        compiler_params=pltpu.CompilerParams(use_tc_tiling_on_sc=True,
                                             needs_layout_passes=False),
        out_type=jax.ShapeDtypeStruct((_B, _VP), jnp.float32),
        scratch_types=[
            pltpu.VMEM((_L, _VP), jnp.float32),
            pltpu.VMEM((_L, _VP), jnp.float32),
            pltpu.VMEM((_L, _SP), jnp.float32),
            pltpu.VMEM((_L, _SP), jnp.float32),
            pltpu.VMEM((_L, _SP), jnp.int32),
            pltpu.VMEM((_L, _SP), jnp.int32),
            pltpu.SemaphoreType.DMA,
            pltpu.SemaphoreType.DMA,
            pltpu.SemaphoreType.DMA,
            pltpu.SemaphoreType.DMA,
        ],
    )(_sc_body)
    acc = sc_call(attn_p, vocab_p, chars_p)

    copy_prob = pl.pallas_call(
        _gate_body,
        out_shape=jax.ShapeDtypeStruct((_B, 1), jnp.float32),
    )(decoder_hidden, context_vector, W1,
      b1.reshape(1, _DEC), jnp.pad(W2, ((0, 127), (0, 0))), b2.reshape(1, 1))

    final = pl.pallas_call(
        _combine_body,
        out_shape=jax.ShapeDtypeStruct((_B, _V), jnp.float32),
    )(vocab_distribution, acc, copy_prob)
    return final, copy_prob


# speedup vs baseline: 1.2914x; 1.0671x over previous
import functools

import jax
import jax.numpy as jnp
from jax import lax
from jax.experimental import pallas as pl
from jax.experimental.pallas import tpu as pltpu
from jax.experimental.pallas import tpu_sc as plsc

_B = 1024
_SRC = 200
_DEC = 512
_ENC = 512
_V = 1000

_L = 16
_NC = 2
_NS = 16
_NW = _NC * _NS
_GPW = _B // _L // _NW
_VP = 1024
_SP = 256


def _gate_body(dh_ref, cv_ref, w1_ref, b1_ref, w2_ref, b2_ref, p_ref):
    w1 = w1_ref[...]
    h = lax.dot_general(dh_ref[...], w1[:, :_DEC], (((1,), (1,)), ((), ())),
                        preferred_element_type=jnp.float32)
    h += lax.dot_general(cv_ref[...], w1[:, _DEC:], (((1,), (1,)), ((), ())),
                         preferred_element_type=jnp.float32)
    h = jnp.tanh(h + b1_ref[...])
    z = lax.dot_general(h, w2_ref[...], (((1,), (1,)), ((), ())),
                        preferred_element_type=jnp.float32)
    p_ref[...] = jax.nn.sigmoid(z[:, :1] + b2_ref[0, 0])


def _combine_body(vocab_ref, acc_ref, p_ref, out_ref):
    vocab = vocab_ref[...]
    acc = acc_ref[...]
    p = p_ref[...]
    vsum = jnp.sum(vocab, axis=1, keepdims=True)
    accsum = jnp.sum(acc, axis=1, keepdims=True)
    total = (1.0 - p) * vsum + p * (accsum - vsum)
    inv = 1.0 / (total + 1e-10)
    out_ref[...] = (vocab * (1.0 - 2.0 * p) + p * acc) * inv


def _sc_body(attn_hbm, vocab_hbm, chars_hbm, acc_hbm,
             vocab_0, vocab_1, attn_0, attn_1, chars_0, chars_1,
             sem_in0, sem_in1, sem_out0, sem_out1):
    wid = lax.axis_index("s") * _NC + lax.axis_index("c")
    iota = lax.iota(jnp.int32, _L)

    sets = ((vocab_0, attn_0, chars_0, sem_in0, sem_out0),
            (vocab_1, attn_1, chars_1, sem_in1, sem_out1))

    in_handles = []
    for k in range(_GPW):
        base = (wid * _GPW + k) * _L
        vocab_v, attn_v, chars_v, sem_in, _ = sets[k]
        in_handles.append([
            pltpu.async_copy(vocab_hbm.at[pl.ds(base, _L), :], vocab_v,
                             sem_in),
            pltpu.async_copy(attn_hbm.at[pl.ds(base, _L), :], attn_v, sem_in),
            pltpu.async_copy(chars_hbm.at[pl.ds(base, _L), :], chars_v,
                             sem_in),
        ])

    out_handles = []
    for k in range(_GPW):
        base = (wid * _GPW + k) * _L
        vocab_v, attn_v, chars_v, sem_in, sem_out = sets[k]
        for h in in_handles[k]:
            h.wait()

        def _scat(s, c, chars_v=chars_v, attn_v=attn_v, vocab_v=vocab_v):
            col = jnp.full((_L,), s, jnp.int32)
            ch = plsc.load_gather(chars_v, [iota, col])
            aw = plsc.load_gather(attn_v, [iota, col])
            plsc.addupdate_scatter(vocab_v, [iota, ch], aw)
            return c
        lax.fori_loop(0, _SRC, _scat, 0, unroll=8)

        out_handles.append(
            pltpu.async_copy(vocab_v, acc_hbm.at[pl.ds(base, _L), :],
                             sem_out))

    for h in out_handles:
        h.wait()


def kernel(decoder_hidden, context_vector, encoder_outputs, attention_weights,
           vocab_distribution, source_chars, W1, b1, W2, b2):
    del encoder_outputs

    chars = source_chars.astype(jnp.int32)

    mesh = plsc.VectorSubcoreMesh(core_axis_name="c", subcore_axis_name="s")
    sc_call = functools.partial(
        pl.kernel, mesh=mesh,
        compiler_params=pltpu.CompilerParams(use_tc_tiling_on_sc=True,
                                             needs_layout_passes=False),
        out_type=jax.ShapeDtypeStruct((_B, _V), jnp.float32),
        scratch_types=[
            pltpu.VMEM((_L, _V), jnp.float32),
            pltpu.VMEM((_L, _V), jnp.float32),
            pltpu.VMEM((_L, _SRC), jnp.float32),
            pltpu.VMEM((_L, _SRC), jnp.float32),
            pltpu.VMEM((_L, _SRC), jnp.int32),
            pltpu.VMEM((_L, _SRC), jnp.int32),
            pltpu.SemaphoreType.DMA,
            pltpu.SemaphoreType.DMA,
            pltpu.SemaphoreType.DMA,
            pltpu.SemaphoreType.DMA,
        ],
    )(_sc_body)
    acc = sc_call(attention_weights, vocab_distribution, chars)

    copy_prob = pl.pallas_call(
        _gate_body,
        out_shape=jax.ShapeDtypeStruct((_B, 1), jnp.float32),
    )(decoder_hidden, context_vector, W1,
      b1.reshape(1, _DEC), jnp.pad(W2, ((0, 127), (0, 0))), b2.reshape(1, 1))

    final = pl.pallas_call(
        _combine_body,
        out_shape=jax.ShapeDtypeStruct((_B, _V), jnp.float32),
    )(vocab_distribution, acc, copy_prob)
    return final, copy_prob


# speedup vs baseline: 1.3588x; 1.0522x over previous
import functools

import jax
import jax.numpy as jnp
from jax import lax
from jax.experimental import pallas as pl
from jax.experimental.pallas import tpu as pltpu
from jax.experimental.pallas import tpu_sc as plsc

_B = 1024
_SRC = 200
_DEC = 512
_ENC = 512
_V = 1000

_L = 16
_NC = 2
_NS = 16
_NW = _NC * _NS
_GPW = _B // _L // _NW
_VP = 1024
_SP = 256


def _gate_body(dh_ref, cv_ref, w1_ref, b1_ref, w2_ref, b2_ref, p_ref):
    w1 = w1_ref[...]
    h = lax.dot_general(dh_ref[...], w1[:, :_DEC], (((1,), (1,)), ((), ())),
                        preferred_element_type=jnp.float32)
    h += lax.dot_general(cv_ref[...], w1[:, _DEC:], (((1,), (1,)), ((), ())),
                         preferred_element_type=jnp.float32)
    h = jnp.tanh(h + b1_ref[...])
    z = lax.dot_general(h, w2_ref[...], (((1,), (1,)), ((), ())),
                        preferred_element_type=jnp.float32)
    p_ref[...] = jax.nn.sigmoid(z[:, :1] + b2_ref[0, 0])


def _combine_body(vocab_ref, acc_ref, p_ref, out_ref):
    vocab = vocab_ref[...]
    acc = acc_ref[...]
    p = p_ref[...]
    vsum = jnp.sum(vocab, axis=1, keepdims=True)
    accsum = jnp.sum(acc, axis=1, keepdims=True)
    total = (1.0 - p) * vsum + p * accsum
    inv = 1.0 / (total + 1e-10)
    out_ref[...] = (vocab * (1.0 - p) + p * acc) * inv


def _sc_body(attn_hbm, chars_hbm, acc_hbm,
             acc_0, acc_1, attn_0, attn_1, chars_0, chars_1,
             sem_in0, sem_in1, sem_out0, sem_out1):
    wid = lax.axis_index("s") * _NC + lax.axis_index("c")
    iota = lax.iota(jnp.int32, _L)
    zeros = jnp.zeros((_L,), jnp.float32)

    sets = ((acc_0, attn_0, chars_0, sem_in0, sem_out0),
            (acc_1, attn_1, chars_1, sem_in1, sem_out1))

    in_handles = []
    for k in range(_GPW):
        base = (wid * _GPW + k) * _L
        _, attn_v, chars_v, sem_in, _ = sets[k]
        in_handles.append([
            pltpu.async_copy(attn_hbm.at[pl.ds(base, _L), :], attn_v, sem_in),
            pltpu.async_copy(chars_hbm.at[pl.ds(base, _L), :], chars_v,
                             sem_in),
        ])

    out_handles = []
    for k in range(_GPW):
        base = (wid * _GPW + k) * _L
        acc_v, attn_v, chars_v, sem_in, sem_out = sets[k]

        def _zrow(r, c, acc_v=acc_v):
            def _z(i, c2):
                acc_v[r, pl.ds(i * _L, _L)] = zeros
                return c2
            lax.fori_loop(0, _V // _L, _z, 0, unroll=8)
            acc_v[r, pl.ds(_V - _L, _L)] = zeros
            return c
        lax.fori_loop(0, _L, _zrow, 0)

        for h in in_handles[k]:
            h.wait()

        def _scat(s, c, chars_v=chars_v, attn_v=attn_v, acc_v=acc_v):
            col = jnp.full((_L,), s, jnp.int32)
            ch = plsc.load_gather(chars_v, [iota, col])
            aw = plsc.load_gather(attn_v, [iota, col])
            plsc.addupdate_scatter(acc_v, [iota, ch], aw)
            return c
        lax.fori_loop(0, _SRC, _scat, 0, unroll=8)

        out_handles.append(
            pltpu.async_copy(acc_v, acc_hbm.at[pl.ds(base, _L), :],
                             sem_out))

    for h in out_handles:
        h.wait()


def kernel(decoder_hidden, context_vector, encoder_outputs, attention_weights,
           vocab_distribution, source_chars, W1, b1, W2, b2):
    del encoder_outputs

    chars = source_chars.astype(jnp.int32)

    mesh = plsc.VectorSubcoreMesh(core_axis_name="c", subcore_axis_name="s")
    sc_call = functools.partial(
        pl.kernel, mesh=mesh,
        compiler_params=pltpu.CompilerParams(use_tc_tiling_on_sc=True,
                                             needs_layout_passes=False),
        out_type=jax.ShapeDtypeStruct((_B, _V), jnp.float32),
        scratch_types=[
            pltpu.VMEM((_L, _V), jnp.float32),
            pltpu.VMEM((_L, _V), jnp.float32),
            pltpu.VMEM((_L, _SRC), jnp.float32),
            pltpu.VMEM((_L, _SRC), jnp.float32),
            pltpu.VMEM((_L, _SRC), jnp.int32),
            pltpu.VMEM((_L, _SRC), jnp.int32),
            pltpu.SemaphoreType.DMA,
            pltpu.SemaphoreType.DMA,
            pltpu.SemaphoreType.DMA,
            pltpu.SemaphoreType.DMA,
        ],
    )(_sc_body)
    acc = sc_call(attention_weights, chars)

    copy_prob = pl.pallas_call(
        _gate_body,
        out_shape=jax.ShapeDtypeStruct((_B, 1), jnp.float32),
    )(decoder_hidden, context_vector, W1,
      b1.reshape(1, _DEC), jnp.pad(W2, ((0, 127), (0, 0))), b2.reshape(1, 1))

    _rb = 128
    final = pl.pallas_call(
        _combine_body,
        grid=(_B // _rb,),
        in_specs=[
            pl.BlockSpec((_rb, _V), lambda i: (i, 0)),
            pl.BlockSpec((_rb, _V), lambda i: (i, 0)),
            pl.BlockSpec((_rb, 1), lambda i: (i, 0)),
        ],
        out_specs=pl.BlockSpec((_rb, _V), lambda i: (i, 0)),
        out_shape=jax.ShapeDtypeStruct((_B, _V), jnp.float32),
    )(vocab_distribution, acc, copy_prob)
    return final, copy_prob


# speedup vs baseline: 1.4669x; 1.0796x over previous
import functools

import jax
import jax.numpy as jnp
from jax import lax
from jax.experimental import pallas as pl
from jax.experimental.pallas import tpu as pltpu
from jax.experimental.pallas import tpu_sc as plsc

_B = 1024
_SRC = 200
_DEC = 512
_ENC = 512
_V = 1000

_L = 16
_NC = 2
_NS = 16
_NW = _NC * _NS
_GPW = _B // _L // _NW
_VP = 1024
_SP = 256


def _gate_body(dh_ref, cv_ref, w1_ref, b1_ref, w2_ref, b2_ref, p_ref):
    w1 = w1_ref[...]
    h = lax.dot_general(dh_ref[...], w1[:, :_DEC], (((1,), (1,)), ((), ())),
                        preferred_element_type=jnp.float32)
    h += lax.dot_general(cv_ref[...], w1[:, _DEC:], (((1,), (1,)), ((), ())),
                         preferred_element_type=jnp.float32)
    h = jnp.tanh(h + b1_ref[...])
    z = lax.dot_general(h, w2_ref[...], (((1,), (1,)), ((), ())),
                        preferred_element_type=jnp.float32)
    p_ref[...] = jax.nn.sigmoid(z + b2_ref[0, 0])


def _combine_body(vocab_ref, acc_ref, p_ref, out_ref):
    vocab = vocab_ref[...]
    acc = acc_ref[...]
    p = p_ref[:, :1]
    vsum = jnp.sum(vocab, axis=1, keepdims=True)
    accsum = jnp.sum(acc, axis=1, keepdims=True)
    total = (1.0 - p) * vsum + p * accsum
    inv = 1.0 / (total + 1e-10)
    out_ref[...] = (vocab * (1.0 - p) + p * acc) * inv


def _sc_body(attn_hbm, chars_hbm, acc_hbm,
             acc_0, acc_1, attn_0, attn_1, chars_0, chars_1,
             sem_in0, sem_in1, sem_out0, sem_out1):
    wid = lax.axis_index("s") * _NC + lax.axis_index("c")
    iota = lax.iota(jnp.int32, _L)
    zeros = jnp.zeros((_L,), jnp.float32)

    sets = ((acc_0, attn_0, chars_0, sem_in0, sem_out0),
            (acc_1, attn_1, chars_1, sem_in1, sem_out1))

    in_handles = []
    for k in range(_GPW):
        base = (wid * _GPW + k) * _L
        _, attn_v, chars_v, sem_in, _ = sets[k]
        in_handles.append([
            pltpu.async_copy(attn_hbm.at[pl.ds(base, _L), :], attn_v, sem_in),
            pltpu.async_copy(chars_hbm.at[pl.ds(base, _L), :], chars_v,
                             sem_in),
        ])

    out_handles = []
    for k in range(_GPW):
        base = (wid * _GPW + k) * _L
        acc_v, attn_v, chars_v, sem_in, sem_out = sets[k]

        def _zrow(r, c, acc_v=acc_v):
            def _z(i, c2):
                acc_v[r, pl.ds(i * _L, _L)] = zeros
                return c2
            lax.fori_loop(0, _V // _L, _z, 0, unroll=8)
            acc_v[r, pl.ds(_V - _L, _L)] = zeros
            return c
        lax.fori_loop(0, _L, _zrow, 0)

        for h in in_handles[k]:
            h.wait()

        def _scat(s, c, chars_v=chars_v, attn_v=attn_v, acc_v=acc_v):
            col = jnp.full((_L,), s, jnp.int32)
            ch = plsc.load_gather(chars_v, [iota, col])
            aw = plsc.load_gather(attn_v, [iota, col])
            plsc.addupdate_scatter(acc_v, [iota, ch], aw)
            return c
        lax.fori_loop(0, _SRC, _scat, 0, unroll=8)

        out_handles.append(
            pltpu.async_copy(acc_v, acc_hbm.at[pl.ds(base, _L), :],
                             sem_out))

    for h in out_handles:
        h.wait()


def kernel(decoder_hidden, context_vector, encoder_outputs, attention_weights,
           vocab_distribution, source_chars, W1, b1, W2, b2):
    del encoder_outputs

    chars = source_chars.astype(jnp.int32)

    mesh = plsc.VectorSubcoreMesh(core_axis_name="c", subcore_axis_name="s")
    sc_call = functools.partial(
        pl.kernel, mesh=mesh,
        compiler_params=pltpu.CompilerParams(use_tc_tiling_on_sc=True,
                                             needs_layout_passes=False),
        out_type=jax.ShapeDtypeStruct((_B, _V), jnp.float32),
        scratch_types=[
            pltpu.VMEM((_L, _V), jnp.float32),
            pltpu.VMEM((_L, _V), jnp.float32),
            pltpu.VMEM((_L, _SRC), jnp.float32),
            pltpu.VMEM((_L, _SRC), jnp.float32),
            pltpu.VMEM((_L, _SRC), jnp.int32),
            pltpu.VMEM((_L, _SRC), jnp.int32),
            pltpu.SemaphoreType.DMA,
            pltpu.SemaphoreType.DMA,
            pltpu.SemaphoreType.DMA,
            pltpu.SemaphoreType.DMA,
        ],
    )(_sc_body)
    acc = sc_call(attention_weights, chars)

    p_wide = pl.pallas_call(
        _gate_body,
        out_shape=jax.ShapeDtypeStruct((_B, 128), jnp.float32),
    )(decoder_hidden, context_vector, W1,
      b1.reshape(1, _DEC), jnp.pad(W2, ((0, 127), (0, 0))), b2.reshape(1, 1))

    _rb = 512
    final = pl.pallas_call(
        _combine_body,
        grid=(_B // _rb,),
        in_specs=[
            pl.BlockSpec((_rb, _V), lambda i: (i, 0)),
            pl.BlockSpec((_rb, _V), lambda i: (i, 0)),
            pl.BlockSpec((_rb, 128), lambda i: (i, 0)),
        ],
        out_specs=pl.BlockSpec((_rb, _V), lambda i: (i, 0)),
        out_shape=jax.ShapeDtypeStruct((_B, _V), jnp.float32),
    )(vocab_distribution, acc, p_wide)
    return final, p_wide[:, :1]
